# baseline (speedup 1.0000x reference)
import functools

import jax
import jax.numpy as jnp
from jax import lax
from jax.experimental import pallas as pl
from jax.experimental.pallas import tpu as pltpu
from jax.experimental.pallas import tpu_sc as plsc

D = 128
S = 512
TBL = 2 * S - 1

NC = 2
NIG = 8
NJG = 4
BI = S // NIG
BJ = S // NJG
WINP = 192
NLANE = 16


@functools.partial(
    pl.kernel,
    out_type=jax.ShapeDtypeStruct((S, NJG, BJ, D), jnp.float32),
    mesh=plsc.VectorSubcoreMesh(core_axis_name="c", subcore_axis_name="s"),
    scratch_types=[
        pltpu.VMEM((128,), jnp.int32),
        pltpu.VMEM((64,), jnp.int32),
        pltpu.VMEM((WINP, D), jnp.float32),
        pltpu.VMEM((BJ, D), jnp.float32),
        pltpu.VMEM((2, BJ, D), jnp.float32),
        pltpu.VMEM((2, BJ, D), jnp.float32),
        pltpu.SemaphoreType.DMA,
        pltpu.SemaphoreType.DMA,
    ],
)
def _sc_rel_pos_add(x_hbm, emb_hbm, out_hbm, idxa_v, idxb_v, winf_v, x_v,
                    resa_v, resb_v, sema, semb):
    c = lax.axis_index("c")
    s = lax.axis_index("s")
    wid = s * NC + c
    ig = wid // NJG
    jg = lax.rem(wid, NJG)
    i0 = ig * BI
    j0 = jg * BJ
    top = i0 - j0 + (S - 1) + (BI - 1)

    xcopy = pltpu.make_async_copy(x_hbm.at[pl.ds(j0, BJ)], x_v, sema)
    xcopy.start()

    iota = lax.iota(jnp.int32, NLANE)
    for k in range(128 // NLANE):
        t0 = k * NLANE
        idxa_v[pl.ds(t0, NLANE)] = jnp.clip(top - t0 - iota, 0, TBL - 1)
    for k in range(64 // NLANE):
        t0 = 128 + k * NLANE
        idxb_v[pl.ds(k * NLANE, NLANE)] = jnp.clip(top - t0 - iota, 0, TBL - 1)

    ga = pltpu.make_async_copy(emb_hbm.at[idxa_v], winf_v.at[pl.ds(0, 128)],
                               semb)
    ga.start()
    gb = pltpu.make_async_copy(emb_hbm.at[idxb_v], winf_v.at[pl.ds(128, 64)],
                               sema)
    gb.start()
    xcopy.wait()
    ga.wait()
    gb.wait()

    def half(q, res_v, sem, dibase):
        @pl.when(q > 0)
        def _wait_prev():
            pltpu.make_async_copy(
                res_v, out_hbm.at[pl.ds(i0 + dibase, 2), jg], sem).wait()

        base = (BI - 1) - dibase
        init = tuple(
            winf_v[base - 1, pl.ds(v * NLANE, NLANE)]
            for v in range(D // NLANE)
        )

        @plsc.parallel_loop(0, BJ, unroll=4, carry=init)
        def _dj(dj, prev):
            w0 = base + dj
            cur = tuple(
                winf_v[w0, pl.ds(v * NLANE, NLANE)]
                for v in range(D // NLANE)
            )
            for v in range(D // NLANE):
                sl = pl.ds(v * NLANE, NLANE)
                xv = x_v[dj, sl]
                res_v[0, dj, sl] = xv + cur[v]
                res_v[1, dj, sl] = xv + prev[v]
            return cur

        pltpu.make_async_copy(
            res_v, out_hbm.at[pl.ds(i0 + dibase, 2), jg], sem).start()

    def q_body(q, carry):
        di0 = 4 * q
        half(q, resa_v, sema, di0)
        half(q, resb_v, semb, di0 + 2)
        return carry

    lax.fori_loop(0, BI // 4, q_body, 0)
    pltpu.make_async_copy(resa_v, out_hbm.at[pl.ds(i0, 2), jg], sema).wait()
    pltpu.make_async_copy(resb_v, out_hbm.at[pl.ds(i0, 2), jg], semb).wait()


def kernel(x, rel_pos_emb):
    xs = x[0]
    out = _sc_rel_pos_add(xs, rel_pos_emb)
    return out.reshape(1, S, S, D)


# speedup vs baseline: 1.0151x; 1.0151x over previous
import functools

import jax
import jax.numpy as jnp
from jax import lax
from jax.experimental import pallas as pl
from jax.experimental.pallas import tpu as pltpu
from jax.experimental.pallas import tpu_sc as plsc

D = 128
S = 512
TBL = 2 * S - 1

NC = 2
NIG = 8
NJG = 4
BI = S // NIG
BJ = S // NJG
WINP = 192
NLANE = 16


@functools.partial(
    pl.kernel,
    out_type=jax.ShapeDtypeStruct((S, NJG, BJ, D), jnp.float32),
    mesh=plsc.VectorSubcoreMesh(core_axis_name="c", subcore_axis_name="s"),
    scratch_types=[
        pltpu.VMEM((128,), jnp.int32),
        pltpu.VMEM((64,), jnp.int32),
        pltpu.VMEM((WINP, D), jnp.float32),
        pltpu.VMEM((BJ, D), jnp.float32),
        pltpu.VMEM((2, BJ, D), jnp.float32),
        pltpu.VMEM((2, BJ, D), jnp.float32),
        pltpu.SemaphoreType.DMA,
        pltpu.SemaphoreType.DMA,
    ],
)
def _sc_rel_pos_add(x_hbm, emb_hbm, out_hbm, idxa_v, idxb_v, winf_v, x_v,
                    resa_v, resb_v, sema, semb):
    c = lax.axis_index("c")
    s = lax.axis_index("s")
    wid = s * NC + c
    ig = wid // NJG
    jg = lax.rem(wid, NJG)
    i0 = ig * BI
    j0 = jg * BJ
    top = i0 - j0 + (S - 1) + (BI - 1)

    xcopy = pltpu.make_async_copy(x_hbm.at[pl.ds(j0, BJ)], x_v, sema)
    xcopy.start()

    iota = lax.iota(jnp.int32, NLANE)
    for k in range(128 // NLANE):
        t0 = k * NLANE
        idxa_v[pl.ds(t0, NLANE)] = jnp.clip(top - t0 - iota, 0, TBL - 1)
    for k in range(64 // NLANE):
        t0 = 128 + k * NLANE
        idxb_v[pl.ds(k * NLANE, NLANE)] = jnp.clip(top - t0 - iota, 0, TBL - 1)

    ga = pltpu.make_async_copy(emb_hbm.at[idxa_v], winf_v.at[pl.ds(0, 128)],
                               semb)
    ga.start()
    gb = pltpu.make_async_copy(emb_hbm.at[idxb_v], winf_v.at[pl.ds(128, 64)],
                               sema)
    gb.start()
    xcopy.wait()
    ga.wait()
    gb.wait()

    def half(q, res_v, sem, dibase):
        @pl.when(q > 0)
        def _wait_prev():
            pltpu.make_async_copy(
                res_v, out_hbm.at[pl.ds(i0 + dibase, 2), jg], sem).wait()

        base = (BI - 1) - dibase
        init = tuple(
            winf_v[base - 1, pl.ds(v * NLANE, NLANE)]
            for v in range(D // NLANE)
        )

        @plsc.parallel_loop(0, BJ, step=2, unroll=2, carry=init)
        def _dj(dj, prev):
            w0 = base + dj
            cur = tuple(
                winf_v[w0, pl.ds(v * NLANE, NLANE)]
                for v in range(D // NLANE)
            )
            nxt = tuple(
                winf_v[w0 + 1, pl.ds(v * NLANE, NLANE)]
                for v in range(D // NLANE)
            )
            for v in range(D // NLANE):
                sl = pl.ds(v * NLANE, NLANE)
                xv = x_v[dj, sl]
                res_v[0, dj, sl] = xv + cur[v]
                res_v[1, dj, sl] = xv + prev[v]
                xw = x_v[dj + 1, sl]
                res_v[0, dj + 1, sl] = xw + nxt[v]
                res_v[1, dj + 1, sl] = xw + cur[v]
            return nxt

        pltpu.make_async_copy(
            res_v, out_hbm.at[pl.ds(i0 + dibase, 2), jg], sem).start()

    def q_body(q, carry):
        di0 = 4 * q
        half(q, resa_v, sema, di0)
        half(q, resb_v, semb, di0 + 2)
        return carry

    lax.fori_loop(0, BI // 4, q_body, 0)
    pltpu.make_async_copy(resa_v, out_hbm.at[pl.ds(i0, 2), jg], sema).wait()
    pltpu.make_async_copy(resb_v, out_hbm.at[pl.ds(i0, 2), jg], semb).wait()


def kernel(x, rel_pos_emb):
    xs = x[0]
    out = _sc_rel_pos_add(xs, rel_pos_emb)
    return out.reshape(1, S, S, D)
